# baseline (speedup 1.0000x reference)
import functools
import numpy as np
import jax
import jax.numpy as jnp
from jax import lax
from jax.experimental import pallas as pl
from jax.experimental.pallas import tpu as pltpu
from jax.experimental.pallas import tpu_sc as plsc

_N = 883
_K = 64
_TID = 10
_DIW = 2
_D = 1 + _K + _TID + _DIW
_TOD_MOD = 12 * 24
_BT = 32 * 12


def _const_table():
    io = np.arange(16, dtype=np.int32)
    c = np.zeros((16, 16), np.int32)
    for r in range(3):
        c[r] = (16 * r + io) // 12
        c[3 + r] = 65 + (16 * r + io) % 12
    c[6] = io
    c[7] = (io < _TID).astype(np.int32)
    c[8] = (io < _DIW).astype(np.int32)
    c[9] = (io < 12).astype(np.int32)
    c[10] = 4
    c[11] = 16
    c[12] = 10
    c[13] = 12
    return c


def _sc_kernel(pairs_per, consts_hbm, tmpl_hbm, x_hbm, tod_hbm, dow_hbm,
               tid_hbm, diw_hbm, out_hbm, slab, xbuf, cbuf, todv, dowv,
               tidrows, diwrows, patch48, sem):
    nc = plsc.get_sparse_core_info().num_cores
    w = lax.axis_index("s") * nc + lax.axis_index("c")

    pltpu.sync_copy(consts_hbm, cbuf)
    pltpu.sync_copy(tmpl_hbm, slab.at[pl.ds(0, _N)])

    pltpu.sync_copy(tod_hbm.at[w], todv)
    pltpu.sync_copy(dow_hbm.at[w], dowv)
    pltpu.async_copy(tid_hbm.at[todv.at[pl.ds(0, 16)]], tidrows, sem).wait()
    pltpu.async_copy(diw_hbm.at[dowv.at[pl.ds(0, 16)]], diwrows, sem).wait()

    row = lambda r: cbuf[r, pl.ds(0, 16)]
    pat_n = [row(r) for r in range(3)]
    pat_c = [row(3 + r) for r in range(3)]
    iota = row(6)
    zeros16 = iota - iota
    m_tid = row(7) != zeros16
    m_diw = row(8) != zeros16
    m_12 = row(9) != zeros16
    four16 = row(10)
    sixteen16 = row(11)
    ten16 = row(12)
    twelve16 = row(13)
    off12 = [twelve16, twelve16 + twelve16, twelve16 + twelve16 + twelve16]

    for j in range(pairs_per):
        p = w * pairs_per + j
        tid_reg = tidrows[j, pl.ds(0, 16)]
        diw_reg = diwrows[j, pl.ds(0, 16)]
        plsc.store_scatter(patch48, [iota], tid_reg, mask=m_tid)
        plsc.store_scatter(patch48, [iota + ten16], diw_reg, mask=m_diw)
        seg = patch48[pl.ds(0, 16)]
        for r in range(3):
            plsc.store_scatter(patch48, [iota + off12[r]], seg, mask=m_12)
        vals = [patch48[pl.ds(16 * r, 16)] for r in range(3)]

        pltpu.sync_copy(x_hbm.at[p], xbuf)

        def temb_body(_, base):
            for r in range(3):
                plsc.store_scatter(slab, [pat_n[r] + base, pat_c[r]],
                                   vals[r])
            return base + four16

        lax.fori_loop(0, 221, temb_body, zeros16)

        def x_body(_, n_idx):
            xv = plsc.load_gather(xbuf, [n_idx])
            plsc.store_scatter(slab, [n_idx, zeros16], xv)
            return n_idx + sixteen16

        lax.fori_loop(0, 56, x_body, iota)

        pltpu.sync_copy(slab.at[pl.ds(0, _N)], out_hbm.at[p])


def kernel(x, t_list, spatial_emb, tid_table, diw_table):
    b, t = x.shape[0], x.shape[1]
    info = plsc.get_sparse_core_info()
    nw = info.num_cores * info.num_subcores
    pairs_per = _BT // nw

    t_idx = t_list.astype(jnp.int32).reshape(_BT)
    tod = (t_idx % _TOD_MOD).reshape(nw, pairs_per)
    dow = ((t_idx // _TOD_MOD) % 7).reshape(nw, pairs_per)
    tod_pad = jnp.pad(tod, ((0, 0), (0, 128 - pairs_per)))
    dow_pad = jnp.pad(dow, ((0, 0), (0, 128 - pairs_per)))
    tmpl = jnp.pad(spatial_emb, ((0, 0), (1, _TID + _DIW)))
    x_pad = jnp.pad(x.reshape(_BT, _N), ((0, 0), (0, 896 - _N)))
    tid_pad = jnp.pad(tid_table, ((0, 0), (0, 128 - _TID)))
    diw_pad = jnp.pad(diw_table, ((0, 0), (0, 128 - _DIW)))
    consts = jnp.pad(jnp.asarray(_const_table()), ((0, 0), (0, 112)))

    mesh = plsc.VectorSubcoreMesh(core_axis_name="c", subcore_axis_name="s")
    kern = functools.partial(
        pl.kernel,
        mesh=mesh,
        compiler_params=pltpu.CompilerParams(needs_layout_passes=False),
        out_type=jax.ShapeDtypeStruct((_BT, _N, _D), jnp.float32),
        scratch_types=[
            pltpu.VMEM((896, _D), jnp.float32),
            pltpu.VMEM((896,), jnp.float32),
            pltpu.VMEM((16, 128), jnp.int32),
            pltpu.VMEM((128,), jnp.int32),
            pltpu.VMEM((128,), jnp.int32),
            pltpu.VMEM((16, 128), jnp.float32),
            pltpu.VMEM((16, 128), jnp.float32),
            pltpu.VMEM((48,), jnp.float32),
            pltpu.SemaphoreType.DMA,
        ],
    )(functools.partial(_sc_kernel, pairs_per))

    out = kern(consts, tmpl, x_pad, tod_pad, dow_pad, tid_pad, diw_pad)
    return out.reshape(b, t, _N, _D)


# speedup vs baseline: 1.0144x; 1.0144x over previous
import functools
import numpy as np
import jax
import jax.numpy as jnp
from jax import lax
from jax.experimental import pallas as pl
from jax.experimental.pallas import tpu as pltpu
from jax.experimental.pallas import tpu_sc as plsc

_N = 883
_K = 64
_TID = 10
_DIW = 2
_D = 1 + _K + _TID + _DIW
_TOD_MOD = 12 * 24
_BT = 32 * 12


def _const_table():
    io = np.arange(16, dtype=np.int32)
    c = np.zeros((16, 16), np.int32)
    for r in range(3):
        c[r] = (16 * r + io) // 12
        c[3 + r] = 65 + (16 * r + io) % 12
    c[6] = io
    c[7] = (io < _TID).astype(np.int32)
    c[8] = (io < _DIW).astype(np.int32)
    c[9] = (io < 12).astype(np.int32)
    c[10] = 4
    c[11] = 16
    c[12] = 10
    c[13] = 12
    return c


def _sc_kernel(pairs_per, consts_hbm, tmpl_hbm, x_hbm, tod_hbm, dow_hbm,
               tid_hbm, diw_hbm, out_hbm, slab, xbuf, cbuf, todv, dowv,
               tidrows, diwrows, patch48, sem, sem_x, sem_out):
    nc = plsc.get_sparse_core_info().num_cores
    w = lax.axis_index("s") * nc + lax.axis_index("c")

    pltpu.sync_copy(consts_hbm, cbuf)
    pltpu.sync_copy(tmpl_hbm, slab.at[pl.ds(0, _N)])

    pltpu.sync_copy(tod_hbm.at[w], todv)
    pltpu.sync_copy(dow_hbm.at[w], dowv)
    pltpu.async_copy(tid_hbm.at[todv.at[pl.ds(0, 16)]], tidrows, sem).wait()
    pltpu.async_copy(diw_hbm.at[dowv.at[pl.ds(0, 16)]], diwrows, sem).wait()

    row = lambda r: cbuf[r, pl.ds(0, 16)]
    pat_n = [row(r) for r in range(3)]
    pat_c = [row(3 + r) for r in range(3)]
    iota = row(6)
    zeros16 = iota - iota
    m_tid = row(7) != zeros16
    m_diw = row(8) != zeros16
    m_12 = row(9) != zeros16
    four16 = row(10)
    sixteen16 = row(11)
    ten16 = row(12)
    twelve16 = row(13)
    off12 = [twelve16, twelve16 + twelve16, twelve16 + twelve16 + twelve16]

    prev_out = None
    for j in range(pairs_per):
        p = w * pairs_per + j
        hx = pltpu.async_copy(x_hbm.at[p], xbuf, sem_x)
        tid_reg = tidrows[j, pl.ds(0, 16)]
        diw_reg = diwrows[j, pl.ds(0, 16)]
        plsc.store_scatter(patch48, [iota], tid_reg, mask=m_tid)
        plsc.store_scatter(patch48, [iota + ten16], diw_reg, mask=m_diw)
        seg = patch48[pl.ds(0, 16)]
        for r in range(3):
            plsc.store_scatter(patch48, [iota + off12[r]], seg, mask=m_12)
        vals = [patch48[pl.ds(16 * r, 16)] for r in range(3)]

        if prev_out is not None:
            prev_out.wait()

        def temb_body(_, base):
            for r in range(3):
                plsc.store_scatter(slab, [pat_n[r] + base, pat_c[r]],
                                   vals[r])
            return base + four16

        lax.fori_loop(0, 221, temb_body, zeros16)
        hx.wait()

        def x_body(_, n_idx):
            xv = plsc.load_gather(xbuf, [n_idx])
            plsc.store_scatter(slab, [n_idx, zeros16], xv)
            return n_idx + sixteen16

        lax.fori_loop(0, 56, x_body, iota)

        prev_out = pltpu.async_copy(slab.at[pl.ds(0, _N)], out_hbm.at[p],
                                    sem_out)
    prev_out.wait()


def kernel(x, t_list, spatial_emb, tid_table, diw_table):
    b, t = x.shape[0], x.shape[1]
    info = plsc.get_sparse_core_info()
    nw = info.num_cores * info.num_subcores
    pairs_per = _BT // nw

    t_idx = t_list.astype(jnp.int32).reshape(_BT)
    tod = (t_idx % _TOD_MOD).reshape(nw, pairs_per)
    dow = ((t_idx // _TOD_MOD) % 7).reshape(nw, pairs_per)
    tod_pad = jnp.pad(tod, ((0, 0), (0, 128 - pairs_per)))
    dow_pad = jnp.pad(dow, ((0, 0), (0, 128 - pairs_per)))
    tmpl = jnp.pad(spatial_emb, ((0, 0), (1, _TID + _DIW)))
    x_pad = jnp.pad(x.reshape(_BT, _N), ((0, 0), (0, 896 - _N)))
    tid_pad = jnp.pad(tid_table, ((0, 0), (0, 128 - _TID)))
    diw_pad = jnp.pad(diw_table, ((0, 0), (0, 128 - _DIW)))
    consts = jnp.pad(jnp.asarray(_const_table()), ((0, 0), (0, 112)))

    mesh = plsc.VectorSubcoreMesh(core_axis_name="c", subcore_axis_name="s")
    kern = functools.partial(
        pl.kernel,
        mesh=mesh,
        compiler_params=pltpu.CompilerParams(needs_layout_passes=False),
        out_type=jax.ShapeDtypeStruct((_BT, _N, _D), jnp.float32),
        scratch_types=[
            pltpu.VMEM((896, _D), jnp.float32),
            pltpu.VMEM((896,), jnp.float32),
            pltpu.VMEM((16, 128), jnp.int32),
            pltpu.VMEM((128,), jnp.int32),
            pltpu.VMEM((128,), jnp.int32),
            pltpu.VMEM((16, 128), jnp.float32),
            pltpu.VMEM((16, 128), jnp.float32),
            pltpu.VMEM((48,), jnp.float32),
            pltpu.SemaphoreType.DMA,
            pltpu.SemaphoreType.DMA,
            pltpu.SemaphoreType.DMA,
        ],
    )(functools.partial(_sc_kernel, pairs_per))

    out = kern(consts, tmpl, x_pad, tod_pad, dow_pad, tid_pad, diw_pad)
    return out.reshape(b, t, _N, _D)
